# baseline (speedup 1.0000x reference)
import functools

import jax
import jax.numpy as jnp
from jax import lax
from jax.experimental import pallas as pl
from jax.experimental.pallas import tpu as pltpu
from jax.experimental.pallas import tpu_sc as plsc

VOCAB = 1000000
EMBED_DIM = 64
BATCH = 16384
HIST = 200

_INFO = plsc.get_sparse_core_info()
NC, NS = _INFO.num_cores, _INFO.num_subcores
NW = NC * NS

B_PER_W = BATCH // NW
CB = 4
CHUNK = CB * HIST
NCHUNKS = B_PER_W // CB
NPAIR = NCHUNKS // 2

assert B_PER_W * NW == BATCH
assert NCHUNKS * CB == B_PER_W
assert NCHUNKS % 2 == 0
assert 2 * (CHUNK + CHUNK * EMBED_DIM) <= 131071


@functools.partial(
    pl.kernel,
    mesh=plsc.VectorSubcoreMesh(core_axis_name="c", subcore_axis_name="s"),
    out_type=jax.ShapeDtypeStruct((BATCH, HIST, EMBED_DIM), jnp.float32),
    scratch_types=[
        pltpu.VMEM((CHUNK,), jnp.int32),
        pltpu.VMEM((CHUNK,), jnp.int32),
        pltpu.VMEM((CHUNK, EMBED_DIM), jnp.float32),
        pltpu.VMEM((CHUNK, EMBED_DIM), jnp.float32),
        pltpu.SemaphoreType.DMA,
        pltpu.SemaphoreType.DMA,
        pltpu.SemaphoreType.DMA,
        pltpu.SemaphoreType.DMA,
    ],
    compiler_params=pltpu.CompilerParams(use_tc_tiling_on_sc=False),
)
def _gather_kernel(idx_hbm, table_hbm, out_hbm,
                   idx0, idx1, rows0, rows1, sg0, sg1, ss0, ss1):
    wid = lax.axis_index("s") * NC + lax.axis_index("c")
    wrow = wid * B_PER_W

    idx_v = (idx0, idx1)
    rows_v = (rows0, rows1)
    sg = (sg0, sg1)
    ss = (ss0, ss1)

    def store_chunk(k, s):
        for j in range(CB):
            pltpu.async_copy(
                rows_v[s].at[pl.ds(j * HIST, HIST)],
                out_hbm.at[wrow + k * CB + j],
                ss[s],
            )

    def wait_store(k, s):
        for j in range(CB):
            pltpu.make_async_copy(
                rows_v[s].at[pl.ds(j * HIST, HIST)],
                out_hbm.at[wrow + k * CB + j],
                ss[s],
            ).wait()

    def half_step(j, s, guard_prev):
        k = 2 * j + s
        o = 1 - s

        def wait_reuse():
            wait_store(k - 2, s)

        def finish_prev():
            pltpu.make_async_copy(
                table_hbm.at[idx_v[o]], rows_v[o], sg[o]
            ).wait()
            store_chunk(k - 1, o)

        if guard_prev:
            pl.when(j >= 1)(wait_reuse)
        else:
            wait_reuse()
        for j2 in range(CB):
            pltpu.sync_copy(
                idx_hbm.at[wrow + k * CB + j2],
                idx_v[s].at[pl.ds(j2 * HIST, HIST)],
            )
        pltpu.async_copy(table_hbm.at[idx_v[s]], rows_v[s], sg[s])
        if guard_prev and s == 0:
            pl.when(j >= 1)(finish_prev)
        else:
            finish_prev()

    def body(j, carry):
        half_step(j, 0, guard_prev=True)
        half_step(j, 1, guard_prev=True)
        return carry

    lax.fori_loop(0, NPAIR, body, 0)

    last = NCHUNKS - 1
    pltpu.make_async_copy(table_hbm.at[idx1], rows1, sg1).wait()
    store_chunk(last, 1)
    wait_store(last - 1, 0)
    wait_store(last, 1)


def kernel(input_ids, table):
    return _gather_kernel(input_ids, table)


# speedup vs baseline: 1.3670x; 1.3670x over previous
import functools

import jax
import jax.numpy as jnp
from jax import lax
from jax.experimental import pallas as pl
from jax.experimental.pallas import tpu as pltpu
from jax.experimental.pallas import tpu_sc as plsc

VOCAB = 1000000
EMBED_DIM = 64
BATCH = 16384
HIST = 200

_INFO = plsc.get_sparse_core_info()
NC, NS = _INFO.num_cores, _INFO.num_subcores
NW = NC * NS

B_TOTAL = BATCH * HIST
B_PER_W = B_TOTAL // NW
CHUNK = 256
NCHUNKS = B_PER_W // CHUNK
CPG = 8
NBLK = NCHUNKS // CPG
NSUPER = NBLK // 2
IDS2_COLS = 128
IDS2_ROWS = B_TOTAL // IDS2_COLS
IDROWS_PER_W = B_PER_W // IDS2_COLS

assert B_PER_W * NW == B_TOTAL
assert NCHUNKS * CHUNK == B_PER_W
assert NBLK * CPG == NCHUNKS and NSUPER * 2 == NBLK


@functools.partial(
    pl.kernel,
    mesh=plsc.VectorSubcoreMesh(core_axis_name="c", subcore_axis_name="s"),
    out_type=jax.ShapeDtypeStruct((B_TOTAL, 128), jnp.float32),
    scratch_types=[
        pltpu.VMEM((2 * CPG, 128), jnp.int32),
        pltpu.VMEM((2 * CPG, 128), jnp.int32),
        pltpu.VMEM((CHUNK, 128), jnp.float32),
        pltpu.VMEM((CHUNK, 128), jnp.float32),
        pltpu.SemaphoreType.DMA,
        pltpu.SemaphoreType.DMA,
        pltpu.SemaphoreType.DMA,
        pltpu.SemaphoreType.DMA,
        pltpu.SemaphoreType.DMA,
        pltpu.SemaphoreType.DMA,
    ],
)
def _gather_kernel(ids2_hbm, t128_hbm, out_hbm,
                   ib0, ib1, rows0, rows1, sib0, sib1, sg0, sg1, ss0, ss1):
    wid = lax.axis_index("s") * NC + lax.axis_index("c")
    wflat = wid * B_PER_W
    widrow = wid * IDROWS_PER_W

    ib = (ib0, ib1)
    rows = (rows0, rows1)
    sib = (sib0, sib1)
    sg = (sg0, sg1)
    ss = (ss0, ss1)

    def idx_block_copy(g, buf):
        return pltpu.make_async_copy(
            ids2_hbm.at[pl.ds(widrow + g * (2 * CPG), 2 * CPG)], ib[buf],
            sib[buf])

    def fire_gather(k, s, buf, i):
        for r in range(2):
            pltpu.async_copy(
                t128_hbm.at[ib[buf].at[2 * i + r]],
                rows[s].at[pl.ds(r * 128, 128)],
                sg[s])

    def wait_gather(s, buf, i):
        for r in range(2):
            pltpu.make_async_copy(
                t128_hbm.at[ib[buf].at[2 * i + r]],
                rows[s].at[pl.ds(r * 128, 128)],
                sg[s]).wait()

    def store_copy(k, s):
        return pltpu.make_async_copy(
            rows[s], out_hbm.at[pl.ds(wflat + k * CHUNK, CHUNK)], ss[s])

    def chunk_step(jj, g, buf, i, guard_a, guard_c):
        k = g * CPG + i
        s = i % 2

        def wait_reuse():
            store_copy(k - 2, s).wait()

        def finish_prev():
            pi = i - 1
            pbuf = buf if pi >= 0 else 1 - buf
            pii = pi % CPG
            wait_gather(1 - s, pbuf, pii)
            store_copy(k - 1, 1 - s).start()

        if guard_a:
            pl.when(jj >= 1)(wait_reuse)
        else:
            wait_reuse()
        fire_gather(k, s, buf, i)
        if guard_c:
            pl.when(jj >= 1)(finish_prev)
        else:
            finish_prev()

    def group(jj, g, buf, first_of_pair):
        idx_block_copy(g, buf).wait()
        for i in range(CPG):
            guard_a = first_of_pair and i < 2
            guard_c = first_of_pair and i < 1
            chunk_step(jj, g, buf, i, guard_a, guard_c)
            if i == 0:
                if first_of_pair:
                    idx_block_copy(g + 1, 1 - buf).start()
                else:
                    pl.when(jj < NSUPER - 1)(
                        lambda: idx_block_copy(g + 1, 1 - buf).start())

    def body(jj, carry):
        g0 = 2 * jj
        group(jj, g0, 0, True)
        group(jj, g0 + 1, 1, False)
        return carry

    idx_block_copy(0, 0).start()
    lax.fori_loop(0, NSUPER, body, 0)

    last = NCHUNKS - 1
    wait_gather(1, 1, CPG - 1)
    store_copy(last, 1).start()
    store_copy(last - 1, 0).wait()
    store_copy(last, 1).wait()


def kernel(input_ids, table):
    ids2 = input_ids.reshape(IDS2_ROWS, IDS2_COLS)
    t128 = jnp.pad(table, ((0, 0), (0, 128 - EMBED_DIM)))
    out128 = _gather_kernel(ids2, t128)
    return lax.slice(out128, (0, 0), (B_TOTAL, EMBED_DIM)).reshape(
        BATCH, HIST, EMBED_DIM)


# speedup vs baseline: 1.3688x; 1.0013x over previous
import functools

import jax
import jax.numpy as jnp
from jax import lax
from jax.experimental import pallas as pl
from jax.experimental.pallas import tpu as pltpu
from jax.experimental.pallas import tpu_sc as plsc

VOCAB = 1000000
EMBED_DIM = 64
BATCH = 16384
HIST = 200

_INFO = plsc.get_sparse_core_info()
NC, NS = _INFO.num_cores, _INFO.num_subcores
NW = NC * NS

B_TOTAL = BATCH * HIST
B_PER_W = B_TOTAL // NW
CHUNK = 128
NCHUNKS = B_PER_W // CHUNK
NSLOT = 4
CPB = 8
CPJ = 2 * CPB
NBODY = NCHUNKS // CPJ
IDS2_COLS = 128
IDS2_ROWS = B_TOTAL // IDS2_COLS
IDROWS_PER_W = B_PER_W // IDS2_COLS

assert B_PER_W * NW == B_TOTAL
assert NCHUNKS * CHUNK == B_PER_W
assert NBODY * CPJ == NCHUNKS


@functools.partial(
    pl.kernel,
    mesh=plsc.VectorSubcoreMesh(core_axis_name="c", subcore_axis_name="s"),
    out_type=jax.ShapeDtypeStruct((B_TOTAL, 128), jnp.float32),
    scratch_types=[
        pltpu.VMEM((CPB, 128), jnp.int32),
        pltpu.VMEM((CPB, 128), jnp.int32),
        pltpu.VMEM((CHUNK, 128), jnp.float32),
        pltpu.VMEM((CHUNK, 128), jnp.float32),
        pltpu.VMEM((CHUNK, 128), jnp.float32),
        pltpu.VMEM((CHUNK, 128), jnp.float32),
        pltpu.SemaphoreType.DMA,
        pltpu.SemaphoreType.DMA,
        pltpu.SemaphoreType.DMA,
        pltpu.SemaphoreType.DMA,
        pltpu.SemaphoreType.DMA,
        pltpu.SemaphoreType.DMA,
        pltpu.SemaphoreType.DMA,
        pltpu.SemaphoreType.DMA,
        pltpu.SemaphoreType.DMA,
        pltpu.SemaphoreType.DMA,
    ],
)
def _gather_kernel(ids2_hbm, t128_hbm, out_hbm, ib0, ib1,
                   r0, r1, r2, r3, sib0, sib1,
                   sg0, sg1, sg2, sg3, ss0, ss1, ss2, ss3):
    wid = lax.axis_index("s") * NC + lax.axis_index("c")
    wflat = wid * B_PER_W
    widrow = wid * IDROWS_PER_W

    ib = (ib0, ib1)
    rows = (r0, r1, r2, r3)
    sib = (sib0, sib1)
    sg = (sg0, sg1, sg2, sg3)
    ss = (ss0, ss1, ss2, ss3)

    def idx_block_copy(blk, buf):
        return pltpu.make_async_copy(
            ids2_hbm.at[pl.ds(widrow + blk * CPB, CPB)], ib[buf], sib[buf])

    def gather_copy(k, s, buf, row):
        return pltpu.make_async_copy(
            t128_hbm.at[ib[buf].at[row]], rows[s], sg[s])

    def store_copy(k, s):
        return pltpu.make_async_copy(
            rows[s], out_hbm.at[pl.ds(wflat + k * CHUNK, CHUNK)], ss[s])

    def chunk_step(jj, i):
        k = jj * CPJ + i
        s = i % NSLOT
        buf = (i // CPB) % 2

        def wait_reuse():
            store_copy(k - NSLOT, s).wait()

        def finish_prev():
            pk = k - 2
            pi = i - 2
            ps = pi % NSLOT
            pbuf = ((pi % CPJ) // CPB) % 2
            prow = pi % CPB
            gather_copy(pk, ps, pbuf, prow).wait()
            store_copy(pk, ps).start()

        if i < NSLOT:
            pl.when(jj >= 1)(wait_reuse)
        else:
            wait_reuse()
        gather_copy(k, s, buf, i % CPB).start()
        if i < 2:
            pl.when(jj >= 1)(finish_prev)
        else:
            finish_prev()

    def body(jj, carry):
        idx_block_copy(2 * jj, 0).wait()
        for i in range(CPB):
            chunk_step(jj, i)
            if i == 1:
                idx_block_copy(2 * jj + 1, 1).start()
        idx_block_copy(2 * jj + 1, 1).wait()
        for i in range(CPB, CPJ):
            chunk_step(jj, i)
            if i == CPB + 1:
                pl.when(jj < NBODY - 1)(
                    lambda: idx_block_copy(2 * jj + 2, 0).start())
        return carry

    idx_block_copy(0, 0).start()
    lax.fori_loop(0, NBODY, body, 0)

    last = NCHUNKS - 1
    for k in (last - 1, last):
        i = k % CPJ
        gather_copy(k, i % NSLOT, (i // CPB) % 2, i % CPB).wait()
        store_copy(k, i % NSLOT).start()
    for k in range(last - 3, last + 1):
        store_copy(k, (k % CPJ) % NSLOT).wait()


def kernel(input_ids, table):
    ids2 = input_ids.reshape(IDS2_ROWS, IDS2_COLS)
    t128 = jnp.pad(table, ((0, 0), (0, 128 - EMBED_DIM)))
    out128 = _gather_kernel(ids2, t128)
    return lax.slice(out128, (0, 0), (B_TOTAL, EMBED_DIM)).reshape(
        BATCH, HIST, EMBED_DIM)
